# baseline (speedup 1.0000x reference)
import jax
import jax.numpy as jnp
from jax import lax
from jax.experimental import pallas as pl
from jax.experimental.pallas import tpu as pltpu
from jax.experimental.pallas import tpu_sc as plsc

N_VAR = 4
N_POINTS = 49152
D = 128
B = 2
V = 2
P = 4096

NC = 2
NS = 16
NW = NC * NS
ROWS_PER_W = (B * V * P) // NW
CH = 64
NCH = ROWS_PER_W // CH


def _gather_body(table_hbm, var_hbm, patch_hbm, out_hbm, *scr):
    idxs = scr[:NCH]
    rows = scr[NCH:2 * NCH]
    var_v = scr[2 * NCH]
    gsem = scr[2 * NCH + 1:3 * NCH + 1]
    sem_w = scr[3 * NCH + 1]
    c = lax.axis_index("c")
    s = lax.axis_index("s")
    w = s * NC + c
    pair = w // 8
    b = pair // V
    v = pair % V
    pbase = (w % 8) * ROWS_PER_W

    stages = [
        pltpu.async_copy(patch_hbm.at[b, pl.ds(pbase + j * CH, CH)], idxs[j],
                         gsem[j])
        for j in range(NCH)
    ]
    pltpu.sync_copy(var_hbm.at[pair], var_v)

    off = var_v[...] * N_POINTS

    def fire_writeback(j):
        return pltpu.async_copy(
            rows[j], out_hbm.at[b, v, 0, pl.ds(pbase + j * CH, CH), :], sem_w)

    gathers = [None] * NCH
    wbs = []
    for j in range(NCH):
        stages[j].wait()
        for i in range(CH // 16):
            sl = pl.ds(i * 16, 16)
            idxs[j][sl] = idxs[j][sl] + off
        gathers[j] = pltpu.async_copy(table_hbm.at[idxs[j]], rows[j], gsem[j])
        if j >= 1:
            gathers[j - 1].wait()
            wbs.append(fire_writeback(j - 1))
    gathers[NCH - 1].wait()
    wbs.append(fire_writeback(NCH - 1))
    for wb in wbs:
        wb.wait()


def kernel(mg_embedding, var_indices, patch_idx):
    table2d = mg_embedding.reshape(N_VAR * N_POINTS, D)
    var_tab = jnp.broadcast_to(
        var_indices.astype(jnp.int32).reshape(B * V, 1), (B * V, 16))

    run = pl.kernel(
        _gather_body,
        out_type=jax.ShapeDtypeStruct((B, V, 1, P, D), jnp.float32),
        mesh=plsc.VectorSubcoreMesh(core_axis_name="c", subcore_axis_name="s"),
        scratch_types=(
            [pltpu.VMEM((CH,), jnp.int32) for _ in range(NCH)]
            + [pltpu.VMEM((CH, D), jnp.float32) for _ in range(NCH)]
            + [pltpu.VMEM((16,), jnp.int32)]
            + [pltpu.SemaphoreType.DMA for _ in range(NCH)]
            + [pltpu.SemaphoreType.DMA]
        ),
    )
    return run(table2d, var_tab, patch_idx.astype(jnp.int32))


# speedup vs baseline: 1.0558x; 1.0558x over previous
import jax
import jax.numpy as jnp
from jax import lax
from jax.experimental import pallas as pl
from jax.experimental.pallas import tpu as pltpu
from jax.experimental.pallas import tpu_sc as plsc

N_VAR = 4
N_POINTS = 49152
D = 128
B = 2
V = 2
P = 4096

NC = 2
NS = 16
NW = NC * NS
ROWS_PER_W = (B * V * P) // NW
CH = 128
NCH = ROWS_PER_W // CH


def _gather_body(table_hbm, var_hbm, patch_hbm, out_hbm, *scr):
    idxs = scr[:NCH]
    rows_v = scr[NCH]
    var_v = scr[NCH + 1]
    gsem = scr[NCH + 2:2 * NCH + 2]
    c = lax.axis_index("c")
    s = lax.axis_index("s")
    w = s * NC + c
    pair = w // 8
    b = pair // V
    v = pair % V
    chunk = w % 8
    pbase = chunk * ROWS_PER_W

    stages = [
        pltpu.async_copy(patch_hbm.at[b, pl.ds(pbase + j * CH, CH)], idxs[j],
                         gsem[j])
        for j in range(NCH)
    ]
    pltpu.sync_copy(var_hbm.at[pair], var_v)

    off = var_v[...] * N_POINTS

    gathers = []
    for j in range(NCH):
        stages[j].wait()
        for i in range(CH // 16):
            sl = pl.ds(i * 16, 16)
            idxs[j][sl] = idxs[j][sl] + off
        gathers.append(
            pltpu.async_copy(table_hbm.at[idxs[j]], rows_v.at[j], gsem[j]))
    for g in gathers:
        g.wait()

    pltpu.sync_copy(rows_v, out_hbm.at[b, v, 0, pl.ds(chunk * NCH, NCH)])


def kernel(mg_embedding, var_indices, patch_idx):
    table2d = mg_embedding.reshape(N_VAR * N_POINTS, D)
    var_tab = jnp.broadcast_to(
        var_indices.astype(jnp.int32).reshape(B * V, 1), (B * V, 16))

    run = pl.kernel(
        _gather_body,
        out_type=jax.ShapeDtypeStruct((B, V, 1, P // CH, CH, D), jnp.float32),
        mesh=plsc.VectorSubcoreMesh(core_axis_name="c", subcore_axis_name="s"),
        scratch_types=(
            [pltpu.VMEM((CH,), jnp.int32) for _ in range(NCH)]
            + [pltpu.VMEM((NCH, CH, D), jnp.float32)]
            + [pltpu.VMEM((16,), jnp.int32)]
            + [pltpu.SemaphoreType.DMA for _ in range(NCH)]
        ),
    )
    out = run(table2d, var_tab, patch_idx.astype(jnp.int32))
    return out.reshape(B, V, 1, P, D)
